# baseline (speedup 1.0000x reference)
import functools
import math

import jax
import jax.numpy as jnp
from jax import lax
from jax.experimental import pallas as pl
from jax.experimental.pallas import tpu as pltpu
from jax.experimental.pallas import tpu_sc as plsc

_VOCAB = 32000
_PAD = 0
_SMOOTHING = 0.1
_CONF = 1.0 - _SMOOTHING
_EPS = _SMOOTHING / (_VOCAB - 2)
_K_CONST = _CONF * math.log(_CONF) + _SMOOTHING * math.log(_EPS)

_N = 4096
_BR = 128
_BC = 3200

_NC = 2
_NS = 16
_NW = _NC * _NS
_L = 16
_BPW = _N // _NW
_GW = 128
_ROWS_PER_TOKEN = _VOCAB // _GW


def _tc_body(t_ref, x_ref, out_ref):
    i = pl.program_id(0)
    j = pl.program_id(1)

    @pl.when(jnp.logical_and(i == 0, j == 0))
    def _init():
        out_ref[0, 0] = 0.0

    x = x_ref[...]
    t = t_ref[0, 0, :]
    validf = (t != _PAD).astype(jnp.float32)

    s_rows = jnp.sum(jnp.sum(x, axis=1) * validf)
    s_first = jnp.sum(x[:, 0] * validf) * _EPS + jnp.sum(validf) * _K_CONST
    extra = jnp.where(j == 0, s_first, 0.0)

    out_ref[0, 0] += extra - _EPS * s_rows


def _sc_body(pred_hbm, tgt_hbm, out_hbm, t_v, idx_v, rows_v, acc_v, sem):
    wid = lax.axis_index("s") * _NC + lax.axis_index("c")
    base = wid * _BPW
    pltpu.sync_copy(tgt_hbm.at[pl.ds(base, _BPW)], t_v)

    lanes = lax.iota(jnp.int32, _L)
    for k in range(_BPW // _L):
        t16 = t_v[pl.ds(k * _L, _L)]
        row = (base + k * _L + lanes) * _ROWS_PER_TOKEN + (t16 >> 7)
        idx_v[pl.ds(k * _L, _L)] = row
    pltpu.async_copy(pred_hbm.at[idx_v], rows_v, sem).wait()

    acc = jnp.zeros((_L,), jnp.float32)
    for k in range(_BPW // _L):
        t16 = t_v[pl.ds(k * _L, _L)]
        g = plsc.load_gather(rows_v, [k * _L + lanes, t16 & (_GW - 1)])
        acc = acc + jnp.where(t16 != _PAD, g, 0.0)
    acc_v[...] = acc
    pltpu.sync_copy(acc_v, out_hbm.at[wid])


_sc_gather = functools.partial(
    pl.kernel,
    mesh=plsc.VectorSubcoreMesh(core_axis_name="c", subcore_axis_name="s"),
    compiler_params=pltpu.CompilerParams(needs_layout_passes=False),
    out_type=jax.ShapeDtypeStruct((_NW, _L), jnp.float32),
    scratch_types=[
        pltpu.VMEM((_BPW,), jnp.int32),
        pltpu.VMEM((_BPW,), jnp.int32),
        pltpu.VMEM((_BPW, _GW), jnp.float32),
        pltpu.VMEM((_L,), jnp.float32),
        pltpu.SemaphoreType.DMA,
    ],
)(_sc_body)


def kernel(pred, target):
    t = target.astype(jnp.int32)
    sc_partials = _sc_gather(pred.reshape(-1, _GW), t)

    t3 = t.reshape(_N // _BR, 1, _BR)
    tc_out = pl.pallas_call(
        _tc_body,
        grid=(_N // _BR, _VOCAB // _BC),
        in_specs=[
            pl.BlockSpec((1, 1, _BR), lambda i, j: (i, 0, 0)),
            pl.BlockSpec((_BR, _BC), lambda i, j: (i, j)),
        ],
        out_specs=pl.BlockSpec(memory_space=pltpu.SMEM),
        out_shape=jax.ShapeDtypeStruct((1, 1), jnp.float32),
        compiler_params=pltpu.CompilerParams(
            dimension_semantics=("arbitrary", "arbitrary"),
        ),
    )(t3, pred)

    return tc_out[0, 0] - (_CONF - _EPS) * jnp.sum(sc_partials)


# speedup vs baseline: 2.2609x; 2.2609x over previous
import math

import jax
import jax.numpy as jnp
from jax.experimental import pallas as pl
from jax.experimental.pallas import tpu as pltpu

_VOCAB = 32000
_PAD = 0
_SMOOTHING = 0.1
_CONF = 1.0 - _SMOOTHING
_EPS = _SMOOTHING / (_VOCAB - 2)
_K_CONST = _CONF * math.log(_CONF) + _SMOOTHING * math.log(_EPS)

_N = 4096
_BR = 128
_BC = 3200
_LW = 128


def _tc_body(t_ref, x_ref, out_ref, acc_ref):
    i = pl.program_id(0)
    j = pl.program_id(1)
    ni = pl.num_programs(0)
    nj = pl.num_programs(1)

    @pl.when(jnp.logical_and(i == 0, j == 0))
    def _init():
        acc_ref[...] = jnp.zeros_like(acc_ref)

    x = x_ref[...]
    t = t_ref[0, 0, :]
    validf = (t != _PAD).astype(jnp.float32)[:, None]

    lane = jax.lax.broadcasted_iota(jnp.int32, (_BR, _LW), 1)
    toff = t[:, None] - j * _BC

    rs_parts = []
    g_parts = []
    for w in range(_BC // _LW):
        xw = x[:, w * _LW:(w + 1) * _LW]
        rs_parts.append(xw)
        g_parts.append(jnp.where(toff - w * _LW == lane, xw, 0.0))

    def _tree(parts):
        while len(parts) > 1:
            nxt = [parts[k] + parts[k + 1] for k in range(0, len(parts) - 1, 2)]
            if len(parts) % 2:
                nxt.append(parts[-1])
            parts = nxt
        return parts[0]

    rs = _tree(rs_parts)
    g = _tree(g_parts)

    first = jnp.where(jnp.logical_and(lane == 0, j == 0),
                      _EPS * x[:, :_LW] + _K_CONST, 0.0)

    acc_ref[...] += (first - _EPS * rs - (_CONF - _EPS) * g) * validf

    @pl.when(jnp.logical_and(i == ni - 1, j == nj - 1))
    def _fin():
        out_ref[0, 0] = jnp.sum(acc_ref[...])


def kernel(pred, target):
    t3 = target.astype(jnp.int32).reshape(_N // _BR, 1, _BR)
    out = pl.pallas_call(
        _tc_body,
        grid=(_N // _BR, _VOCAB // _BC),
        in_specs=[
            pl.BlockSpec((1, 1, _BR), lambda i, j: (i, 0, 0)),
            pl.BlockSpec((_BR, _BC), lambda i, j: (i, j)),
        ],
        out_specs=pl.BlockSpec(memory_space=pltpu.SMEM),
        out_shape=jax.ShapeDtypeStruct((1, 1), jnp.float32),
        scratch_shapes=[pltpu.VMEM((_BR, _LW), jnp.float32)],
        compiler_params=pltpu.CompilerParams(
            dimension_semantics=("arbitrary", "arbitrary"),
        ),
    )(t3, pred)
    return out[0, 0]


# speedup vs baseline: 3.6549x; 1.6165x over previous
import math

import jax
import jax.numpy as jnp
from jax.experimental import pallas as pl
from jax.experimental.pallas import tpu as pltpu

_VOCAB = 32000
_PAD = 0
_SMOOTHING = 0.1
_CONF = 1.0 - _SMOOTHING
_EPS = _SMOOTHING / (_VOCAB - 2)
_K_CONST = _CONF * math.log(_CONF) + _SMOOTHING * math.log(_EPS)

_N = 4096
_BR = 64
_BC = 32000
_LW = 128


def _tc_body(t_ref, x_ref, out_ref, acc_ref):
    i = pl.program_id(0)
    j = pl.program_id(1)
    ni = pl.num_programs(0)
    nj = pl.num_programs(1)

    @pl.when(jnp.logical_and(i == 0, j == 0))
    def _init():
        acc_ref[...] = jnp.zeros_like(acc_ref)

    x = x_ref[...]
    t = t_ref[0, 0, :]
    validf = (t != _PAD).astype(jnp.float32)[:, None]

    lane = jax.lax.broadcasted_iota(jnp.int32, (_BR, _LW), 1)
    toff = t[:, None] - j * _BC

    rs_parts = []
    g_parts = []
    for w in range(_BC // _LW):
        xw = x[:, w * _LW:(w + 1) * _LW]
        rs_parts.append(xw)
        g_parts.append(jnp.where(toff - w * _LW == lane, xw, 0.0))

    def _tree(parts):
        while len(parts) > 1:
            nxt = [parts[k] + parts[k + 1] for k in range(0, len(parts) - 1, 2)]
            if len(parts) % 2:
                nxt.append(parts[-1])
            parts = nxt
        return parts[0]

    rs = _tree(rs_parts)
    g = _tree(g_parts)

    first = jnp.where(jnp.logical_and(lane == 0, j == 0),
                      _EPS * x[:, :_LW] + _K_CONST, 0.0)

    acc_ref[...] += (first - _EPS * rs - (_CONF - _EPS) * g) * validf

    @pl.when(jnp.logical_and(i == ni - 1, j == nj - 1))
    def _fin():
        out_ref[0, 0] = jnp.sum(acc_ref[...])


def kernel(pred, target):
    t3 = target.astype(jnp.int32).reshape(_N // _BR, 1, _BR)
    out = pl.pallas_call(
        _tc_body,
        grid=(_N // _BR, _VOCAB // _BC),
        in_specs=[
            pl.BlockSpec((1, 1, _BR), lambda i, j: (i, 0, 0)),
            pl.BlockSpec((_BR, _BC), lambda i, j: (i, j)),
        ],
        out_specs=pl.BlockSpec(memory_space=pltpu.SMEM),
        out_shape=jax.ShapeDtypeStruct((1, 1), jnp.float32),
        scratch_shapes=[pltpu.VMEM((_BR, _LW), jnp.float32)],
        compiler_params=pltpu.CompilerParams(
            dimension_semantics=("arbitrary", "arbitrary"),
        ),
    )(t3, pred)
    return out[0, 0]
